# baseline (speedup 1.0000x reference)
import jax
import jax.numpy as jnp
from jax import lax
from jax.experimental import pallas as pl
from jax.experimental.pallas import tpu as pltpu, tpu_sc as plsc

N_SPIXELS = 2304
NC, NS, L = 2, 16, 16
NW = NC * NS
HW = 384 * 384
NPLANES = 4 * 96
PPW = NPLANES // NW
CH = 1024
NCHUNK = HW // CH
DEPTH = 4


def _body(data_hbm, idx_hbm, out_hbm,
          idx_bufs, dat_bufs, accums, counts, sems):
    wid = lax.axis_index("s") * NC + lax.axis_index("c")
    bufs = tuple(zip(idx_bufs, dat_bufs, sems))

    zeros = jnp.zeros((L,), jnp.float32)
    ones = jnp.ones((L,), jnp.float32)

    def issue(c, b):
        idx_v, dat_v, sem = bufs[b]
        off = c * CH
        pltpu.async_copy(idx_hbm.at[pl.ds(off, CH)], idx_v, sem)
        pltpu.async_copy(data_hbm.at[pl.ds(wid, 1), :, pl.ds(off, CH)],
                         dat_v, sem)

    def wait(b):
        idx_v, dat_v, sem = bufs[b]
        pltpu.make_async_copy(idx_hbm.at[pl.ds(0, CH)], idx_v, sem).wait()
        pltpu.make_async_copy(data_hbm.at[pl.ds(0, 1), :, pl.ds(0, CH)],
                              dat_v, sem).wait()

    def compute(b):
        idx_v, dat_v, _ = bufs[b]

        @plsc.parallel_loop(0, CH // L, unroll=2)
        def _grp(g):
            s = g * L
            iv = idx_v[pl.ds(s, L)]
            plsc.addupdate_scatter(counts, [iv], ones)
            for p in range(PPW):
                x = dat_v[0, p, pl.ds(s, L)]
                plsc.addupdate_scatter(accums[p], [iv], x)

    def zero_acc(i, _):
        s = i * L
        for p in range(PPW):
            accums[p][pl.ds(s, L)] = zeros
        counts[pl.ds(s, L)] = zeros
        return 0
    lax.fori_loop(0, N_SPIXELS // L, zero_acc, 0)

    for d in range(DEPTH):
        issue(d, d)

    def ring_body(h, _):
        c0 = h * DEPTH
        for d in range(DEPTH):
            wait(d)
            compute(d)

            @pl.when(c0 + DEPTH + d < NCHUNK)
            def _():
                issue(c0 + DEPTH + d, d)
        return 0
    lax.fori_loop(0, NCHUNK // DEPTH, ring_body, 0)

    def fin_body(g, _):
        s = g * L
        inv = 1.0 / jnp.maximum(counts[pl.ds(s, L)], 1.0)
        for p in range(PPW):
            accums[p][pl.ds(s, L)] = accums[p][pl.ds(s, L)] * inv
        return 0
    lax.fori_loop(0, N_SPIXELS // L, fin_body, 0)

    for p in range(PPW):
        pltpu.sync_copy(accums[p], out_hbm.at[wid * PPW + p])


@jax.jit
def _spixel_feats(data, idx):
    mesh = plsc.VectorSubcoreMesh(core_axis_name="c", subcore_axis_name="s",
                                  num_cores=NC, num_subcores=NS)
    fn = pl.kernel(
        _body,
        out_type=jax.ShapeDtypeStruct((NPLANES, N_SPIXELS), jnp.float32),
        mesh=mesh,
        compiler_params=pltpu.CompilerParams(needs_layout_passes=False),
        scratch_types=[
            [pltpu.VMEM((CH,), jnp.int32) for _ in range(DEPTH)],
            [pltpu.VMEM((1, PPW, CH), jnp.float32) for _ in range(DEPTH)],
            [pltpu.VMEM((N_SPIXELS,), jnp.float32) for _ in range(PPW)],
            pltpu.VMEM((N_SPIXELS,), jnp.float32),
            [pltpu.SemaphoreType.DMA for _ in range(DEPTH)],
        ],
    )
    return fn(data, idx)


def kernel(pixel_feats, index_map):
    B, C, H, W = pixel_feats.shape
    data = pixel_feats.reshape(NW, PPW, H * W)
    idx = index_map.reshape(-1)
    out = _spixel_feats(data, idx)
    return out.reshape(B, C, N_SPIXELS)


# speedup vs baseline: 1.0056x; 1.0056x over previous
import jax
import jax.numpy as jnp
from jax import lax
from jax.experimental import pallas as pl
from jax.experimental.pallas import tpu as pltpu, tpu_sc as plsc

N_SPIXELS = 2304
NC, NS, L = 2, 16, 16
NW = NC * NS
HW = 384 * 384
NPLANES = 4 * 96
PPW = NPLANES // NW
CH = 2048
NCHUNK = HW // CH
DEPTH = 2


def _body(data_hbm, idx_hbm, out_hbm,
          idx_bufs, dat_bufs, accums, counts, sems):
    wid = lax.axis_index("s") * NC + lax.axis_index("c")
    bufs = tuple(zip(idx_bufs, dat_bufs, sems))

    zeros = jnp.zeros((L,), jnp.float32)
    ones = jnp.ones((L,), jnp.float32)

    def issue(c, b):
        idx_v, dat_v, sem = bufs[b]
        off = c * CH
        pltpu.async_copy(idx_hbm.at[pl.ds(off, CH)], idx_v, sem)
        pltpu.async_copy(data_hbm.at[pl.ds(wid, 1), :, pl.ds(off, CH)],
                         dat_v, sem)

    def wait(b):
        idx_v, dat_v, sem = bufs[b]
        pltpu.make_async_copy(idx_hbm.at[pl.ds(0, CH)], idx_v, sem).wait()
        pltpu.make_async_copy(data_hbm.at[pl.ds(0, 1), :, pl.ds(0, CH)],
                              dat_v, sem).wait()

    def compute(b):
        idx_v, dat_v, _ = bufs[b]

        @plsc.parallel_loop(0, CH // L, unroll=4)
        def _grp(g):
            s = g * L
            iv = idx_v[pl.ds(s, L)]
            plsc.addupdate_scatter(counts, [iv], ones)
            for p in range(PPW):
                x = dat_v[0, p, pl.ds(s, L)]
                plsc.addupdate_scatter(accums[p], [iv], x)

    def zero_acc(i, _):
        s = i * L
        for p in range(PPW):
            accums[p][pl.ds(s, L)] = zeros
        counts[pl.ds(s, L)] = zeros
        return 0
    lax.fori_loop(0, N_SPIXELS // L, zero_acc, 0)

    for d in range(DEPTH):
        issue(d, d)

    def ring_body(h, _):
        c0 = h * DEPTH
        for d in range(DEPTH):
            wait(d)
            compute(d)

            @pl.when(c0 + DEPTH + d < NCHUNK)
            def _():
                issue(c0 + DEPTH + d, d)
        return 0
    lax.fori_loop(0, NCHUNK // DEPTH, ring_body, 0)

    def fin_body(g, _):
        s = g * L
        inv = 1.0 / jnp.maximum(counts[pl.ds(s, L)], 1.0)
        for p in range(PPW):
            accums[p][pl.ds(s, L)] = accums[p][pl.ds(s, L)] * inv
        return 0
    lax.fori_loop(0, N_SPIXELS // L, fin_body, 0)

    for p in range(PPW):
        pltpu.sync_copy(accums[p], out_hbm.at[wid * PPW + p])


@jax.jit
def _spixel_feats(data, idx):
    mesh = plsc.VectorSubcoreMesh(core_axis_name="c", subcore_axis_name="s",
                                  num_cores=NC, num_subcores=NS)
    fn = pl.kernel(
        _body,
        out_type=jax.ShapeDtypeStruct((NPLANES, N_SPIXELS), jnp.float32),
        mesh=mesh,
        compiler_params=pltpu.CompilerParams(needs_layout_passes=False),
        scratch_types=[
            [pltpu.VMEM((CH,), jnp.int32) for _ in range(DEPTH)],
            [pltpu.VMEM((1, PPW, CH), jnp.float32) for _ in range(DEPTH)],
            [pltpu.VMEM((N_SPIXELS,), jnp.float32) for _ in range(PPW)],
            pltpu.VMEM((N_SPIXELS,), jnp.float32),
            [pltpu.SemaphoreType.DMA for _ in range(DEPTH)],
        ],
    )
    return fn(data, idx)


def kernel(pixel_feats, index_map):
    B, C, H, W = pixel_feats.shape
    data = pixel_feats.reshape(NW, PPW, H * W)
    idx = index_map.reshape(-1)
    out = _spixel_feats(data, idx)
    return out.reshape(B, C, N_SPIXELS)
